# baseline (speedup 1.0000x reference)
import jax
import jax.numpy as jnp
from jax.experimental import pallas as pl
import jax.experimental.pallas.tpu as pltpu

USER_N = 8192
ITEM_N = 8192
EMB_D = 32
BI = 512
NI = USER_N // BI


def _lightgcn_kernel(adj_ref, eut_ref, eu_ref, ei_ref,
                     g1u, g2u, g3u, l1u, l2u, l3u,
                     g1i, g2i, g3i, l1i, l2i, l3i,
                     w1u, wi12_t, w3i_t, wi12):
    p = pl.program_id(0)
    i = pl.program_id(1)
    sl = pl.ds(i * BI, BI)
    D = EMB_D

    @pl.when(p == 0)
    def _pass1():
        a = adj_ref[...]
        w1u_i = jax.lax.dot_general(
            a, ei_ref[...], (((1,), (0,)), ((), ())),
            preferred_element_type=jnp.float32)
        w1u[sl, :] = w1u_i
        eu_i = eu_ref[...]
        g1u[...] = w1u_i
        l1u[...] = eu_i + w1u_i
        lhs = jnp.concatenate(
            [eut_ref[:, sl], w1u_i.T], axis=0)
        contrib = jax.lax.dot_general(
            lhs, a, (((1,), (0,)), ((), ())),
            preferred_element_type=jnp.float32)

        @pl.when(i == 0)
        def _():
            wi12_t[...] = contrib

        @pl.when(i > 0)
        def _():
            wi12_t[...] += contrib

    @pl.when((p == 1) & (i == 0))
    def _mid():
        wi12[...] = wi12_t[...].T

    @pl.when(p == 1)
    def _pass2():
        a = adj_ref[...]
        uu = jax.lax.dot_general(
            a, wi12[...], (((1,), (0,)), ((), ())),
            preferred_element_type=jnp.float32)
        w2u_i = uu[:, 0:D]
        w3u_i = uu[:, D:2 * D]
        w1u_i = w1u[sl, :]
        eu_i = eu_ref[...]
        g2u[...] = w1u_i + w2u_i
        g3u[...] = w1u_i + 2.0 * w2u_i + w3u_i
        l2u[...] = eu_i + 2.0 * w1u_i + w2u_i
        l3u[...] = eu_i + 3.0 * w1u_i + 3.0 * w2u_i + w3u_i
        contrib3 = jax.lax.dot_general(
            w2u_i.T, a, (((1,), (0,)), ((), ())),
            preferred_element_type=jnp.float32)

        @pl.when(i == 0)
        def _():
            w3i_t[...] = contrib3

        @pl.when(i > 0)
        def _():
            w3i_t[...] += contrib3

    @pl.when(p == 2)
    def _epilogue():
        w1i = wi12_t[0:D, sl].T
        w2i = wi12_t[D:2 * D, sl].T
        w3i = w3i_t[:, sl].T
        ei = ei_ref[sl, :]
        g1i[...] = w1i
        g2i[...] = w1i + w2i
        g3i[...] = w1i + 2.0 * w2i + w3i
        l1i[...] = ei + w1i
        l2i[...] = ei + 2.0 * w1i + w2i
        l3i[...] = ei + 3.0 * w1i + 3.0 * w2i + w3i


def _run(adj, e_u_t, e_u, e_i):
    D = EMB_D
    out_sd = jax.ShapeDtypeStruct((USER_N, D), jnp.float32)
    out_shape = [out_sd] * 12

    def adj_map(p, i):
        return (jnp.where(p == 2, NI - 1, i), 0)

    def p0_map(p, i):
        return (jnp.where(p == 0, i, NI - 1), 0)

    def p1_map(p, i):
        return (jnp.where(p == 0, 0, jnp.where(p == 1, i, NI - 1)), 0)

    def p2_map(p, i):
        return (jnp.where(p == 2, i, 0), 0)

    return pl.pallas_call(
        _lightgcn_kernel,
        grid=(3, NI),
        in_specs=[
            pl.BlockSpec((BI, ITEM_N), adj_map),
            pl.BlockSpec((D, USER_N), lambda p, i: (0, 0)),
            pl.BlockSpec((BI, D), lambda p, i: (i, 0)),
            pl.BlockSpec((ITEM_N, D), lambda p, i: (0, 0)),
        ],
        out_specs=([pl.BlockSpec((BI, D), m) for m in
                    (p0_map, p1_map, p1_map, p0_map, p1_map, p1_map)]
                   + [pl.BlockSpec((BI, D), p2_map)] * 6),
        out_shape=out_shape,
        scratch_shapes=[
            pltpu.VMEM((USER_N, D), jnp.float32),
            pltpu.VMEM((2 * D, ITEM_N), jnp.float32),
            pltpu.VMEM((D, ITEM_N), jnp.float32),
            pltpu.VMEM((ITEM_N, 2 * D), jnp.float32),
        ],
    )(adj, e_u_t, e_u, e_i)


def kernel(adj, embeds):
    e_u = embeds[:USER_N]
    e_i = embeds[USER_N:]
    e_u_t = e_u.T
    (g1u, g2u, g3u, l1u, l2u, l3u,
     g1i, g2i, g3i, l1i, l2i, l3i) = _run(adj, e_u_t, e_u, e_i)
    lats = (embeds,
            jnp.concatenate([l1u, l1i], axis=0),
            jnp.concatenate([l2u, l2i], axis=0),
            jnp.concatenate([l3u, l3i], axis=0))
    gcn_lats = (embeds,
                jnp.concatenate([g1u, g1i], axis=0),
                jnp.concatenate([g2u, g2i], axis=0),
                jnp.concatenate([g3u, g3i], axis=0))
    return (lats, gcn_lats)
